# baseline (speedup 1.0000x reference)
import jax
import jax.numpy as jnp
from jax import lax
from jax.experimental import pallas as pl
from jax.experimental.pallas import tpu as pltpu
from jax.experimental.pallas import tpu_sc as plsc

B = 16384
HID = 256
OUT = 256
BB = 4096
NB = B // BB

_DIMS = (8, 32, 32, 64, 16)
_OFF = (0, 8, 40, 72, 136, 152, 280)

_NC = 2
_NS = 16
_NW = _NC * _NS
_BPW = B // _NW
_CH = 128
_NCH = _BPW // _CH


def _sc_gather_body(i0, i1, i2, i3, i4, t0, t1, t2, t3, t4,
                    o0, o1, o2, o3, o4,
                    x0, x1, x2, x3, x4, r0, r1, r2, r3, r4, sem):
    wid = lax.axis_index("s") * _NC + lax.axis_index("c")
    base = wid * _BPW
    idxs = (i0, i1, i2, i3, i4)
    tabs = (t0, t1, t2, t3, t4)
    outs = (o0, o1, o2, o3, o4)
    xb = (x0, x1, x2, x3, x4)
    rb = (r0, r1, r2, r3, r4)
    for f in range(5):
        pltpu.sync_copy(idxs[f].at[pl.ds(wid * _NCH, _NCH)], xb[f])
    handles = []
    for f in range(5):
        for j in range(_NCH):
            handles.append(pltpu.async_copy(
                tabs[f].at[xb[f].at[j]],
                rb[f].at[pl.ds(j * _CH, _CH)], sem))
    for h in handles:
        h.wait()
    for f in range(5):
        pltpu.sync_copy(rb[f], outs[f].at[pl.ds(base, _BPW)])


def _sc_gather(down_idx, form_idx, pers_idx, def_idx, sit_idx,
               E_down, E_form, E_pers, E_def, E_sit):
    mesh = plsc.VectorSubcoreMesh(core_axis_name="c", subcore_axis_name="s")
    k = pl.kernel(
        _sc_gather_body,
        [jax.ShapeDtypeStruct((B, d), jnp.float32) for d in _DIMS],
        mesh=mesh,
        compiler_params=pltpu.CompilerParams(use_tc_tiling_on_sc=False),
        scratch_types=(
            [pltpu.VMEM((_NCH, _CH), jnp.int32) for _ in range(5)]
            + [pltpu.VMEM((_BPW, d), jnp.float32) for d in _DIMS]
            + [pltpu.SemaphoreType.DMA]),
    )
    idx2d = [i.reshape(B // _CH, _CH)
             for i in (down_idx, form_idx, pers_idx, def_idx, sit_idx)]
    return k(*idx2d, E_down, E_form, E_pers, E_def, E_sit)


def _dense_body(cont_ref, c0, c1, c2, c3, c4,
                w1_ref, b1_ref, w2_ref, b2_ref, p1_ref, b3_ref, p2_ref,
                b4_ref, out_ref):
    cont = cont_ref[...]
    p1 = p1_ref[...]
    h = jnp.maximum(cont @ w1_ref[...] + b1_ref[...], 0.0)
    ce = h @ w2_ref[...] + b2_ref[...]
    acc = ce @ p1[_OFF[5]:_OFF[6]]
    for f, cref in enumerate((c0, c1, c2, c3, c4)):
        acc = acc + cref[...] @ p1[_OFF[f]:_OFF[f + 1]]
    acc = acc + b3_ref[...]
    out_ref[...] = jnp.maximum(acc, 0.0) @ p2_ref[...] + b4_ref[...]


def kernel(continuous, down_idx, form_idx, pers_idx, def_idx, sit_idx,
           E_down, E_form, E_pers, E_def, E_sit,
           W1, b1, W2, b2, P1, b3, P2, b4):
    cat = _sc_gather(down_idx, form_idx, pers_idx, def_idx, sit_idx,
                     E_down, E_form, E_pers, E_def, E_sit)

    full = lambda shape: pl.BlockSpec(shape, lambda i: (0,) * len(shape))
    grid_spec = pl.GridSpec(
        grid=(NB,),
        in_specs=[
            pl.BlockSpec((BB, 3), lambda i: (i, 0)),
        ] + [
            pl.BlockSpec((BB, d), lambda i: (i, 0)) for d in _DIMS
        ] + [
            full((3, HID)), full((1, HID)),
            full((HID, HID // 2)), full((1, HID // 2)),
            full((280, HID)), full((1, HID)),
            full((HID, OUT)), full((1, OUT)),
        ],
        out_specs=pl.BlockSpec((BB, OUT), lambda i: (i, 0)),
    )
    return pl.pallas_call(
        _dense_body,
        grid_spec=grid_spec,
        out_shape=jax.ShapeDtypeStruct((B, OUT), jnp.float32),
    )(continuous, *cat,
      W1, b1.reshape(1, HID), W2, b2.reshape(1, HID // 2),
      P1, b3.reshape(1, HID), P2, b4.reshape(1, OUT))


# speedup vs baseline: 1.3593x; 1.3593x over previous
import jax
import jax.numpy as jnp
from jax import lax
from jax.experimental import pallas as pl
from jax.experimental.pallas import tpu as pltpu
from jax.experimental.pallas import tpu_sc as plsc

B = 16384
HID = 256
OUT = 256
BB = 4096
NB = B // BB

_DIMS = (8, 32, 32, 64, 16)
_OFF = (0, 8, 40, 72, 136, 152, 280)

_NC = 2
_NS = 16
_NW = _NC * _NS
_BPW = B // _NW
_CH = 128
_NCH = _BPW // _CH


def _sc_gather_body(i0, i1, i2, i3, i4, t0, t1, t2, t3, t4,
                    o0, o1, o2, o3, o4,
                    tv0, tv1, tv2, tv3, tv4,
                    x0, x1, x2, x3, x4, r0, r1, r2, r3, r4):
    wid = lax.axis_index("s") * _NC + lax.axis_index("c")
    base = wid * _BPW
    idxs = (i0, i1, i2, i3, i4)
    tabs = (t0, t1, t2, t3, t4)
    outs = (o0, o1, o2, o3, o4)
    tv = (tv0, tv1, tv2, tv3, tv4)
    xb = (x0, x1, x2, x3, x4)
    rb = (r0, r1, r2, r3, r4)
    for f in range(5):
        pltpu.sync_copy(tabs[f], tv[f])
        pltpu.sync_copy(idxs[f].at[pl.ds(base, _BPW)], xb[f])
    lanes = lax.iota(jnp.int32, 16)

    def step(g, _):
        pos = g * 16 + lanes
        for f in range(5):
            iv = plsc.load_gather(xb[f], [pos])
            for c in range(_DIMS[f]):
                col = jnp.full((16,), c, jnp.int32)
                val = plsc.load_gather(tv[f], [iv, col])
                plsc.store_scatter(rb[f], [pos, col], val)
        return _

    lax.fori_loop(0, _BPW // 16, step, None)
    for f in range(5):
        pltpu.sync_copy(rb[f], outs[f].at[pl.ds(base, _BPW)])


def _sc_gather(down_idx, form_idx, pers_idx, def_idx, sit_idx,
               E_down, E_form, E_pers, E_def, E_sit):
    mesh = plsc.VectorSubcoreMesh(core_axis_name="c", subcore_axis_name="s")
    tables = (E_down, E_form, E_pers, E_def, E_sit)
    k = pl.kernel(
        _sc_gather_body,
        [jax.ShapeDtypeStruct((B, d), jnp.float32) for d in _DIMS],
        mesh=mesh,
        compiler_params=pltpu.CompilerParams(use_tc_tiling_on_sc=False,
                                             needs_layout_passes=False),
        scratch_types=(
            [pltpu.VMEM(t.shape, jnp.float32) for t in tables]
            + [pltpu.VMEM((_BPW,), jnp.int32) for _ in range(5)]
            + [pltpu.VMEM((_BPW, d), jnp.float32) for d in _DIMS]),
    )
    return k(down_idx, form_idx, pers_idx, def_idx, sit_idx, *tables)


def _dense_body(cont_ref, c0, c1, c2, c3, c4,
                w1_ref, b1_ref, w2_ref, b2_ref, p1_ref, b3_ref, p2_ref,
                b4_ref, out_ref):
    cont = cont_ref[...]
    p1 = p1_ref[...]
    h = jnp.maximum(cont @ w1_ref[...] + b1_ref[...], 0.0)
    ce = h @ w2_ref[...] + b2_ref[...]
    acc = ce @ p1[_OFF[5]:_OFF[6]]
    for f, cref in enumerate((c0, c1, c2, c3, c4)):
        acc = acc + cref[...] @ p1[_OFF[f]:_OFF[f + 1]]
    acc = acc + b3_ref[...]
    out_ref[...] = jnp.maximum(acc, 0.0) @ p2_ref[...] + b4_ref[...]


def kernel(continuous, down_idx, form_idx, pers_idx, def_idx, sit_idx,
           E_down, E_form, E_pers, E_def, E_sit,
           W1, b1, W2, b2, P1, b3, P2, b4):
    cat = _sc_gather(down_idx, form_idx, pers_idx, def_idx, sit_idx,
                     E_down, E_form, E_pers, E_def, E_sit)

    full = lambda shape: pl.BlockSpec(shape, lambda i: (0,) * len(shape))
    grid_spec = pl.GridSpec(
        grid=(NB,),
        in_specs=[
            pl.BlockSpec((BB, 3), lambda i: (i, 0)),
        ] + [
            pl.BlockSpec((BB, d), lambda i: (i, 0)) for d in _DIMS
        ] + [
            full((3, HID)), full((1, HID)),
            full((HID, HID // 2)), full((1, HID // 2)),
            full((280, HID)), full((1, HID)),
            full((HID, OUT)), full((1, OUT)),
        ],
        out_specs=pl.BlockSpec((BB, OUT), lambda i: (i, 0)),
    )
    return pl.pallas_call(
        _dense_body,
        grid_spec=grid_spec,
        out_shape=jax.ShapeDtypeStruct((B, OUT), jnp.float32),
    )(continuous, *cat,
      W1, b1.reshape(1, HID), W2, b2.reshape(1, HID // 2),
      P1, b3.reshape(1, HID), P2, b4.reshape(1, OUT))


# speedup vs baseline: 1.4201x; 1.0447x over previous
import jax
import jax.numpy as jnp
from jax import lax
from jax.experimental import pallas as pl
from jax.experimental.pallas import tpu as pltpu
from jax.experimental.pallas import tpu_sc as plsc

B = 16384
HID = 256
OUT = 256
BB = 4096
NB = B // BB

_DIMS = (8, 32, 32, 64, 16)
_OFF = (0, 8, 40, 72, 136, 152, 280)

_NC = 2
_NS = 16
_NW = _NC * _NS
_BPW = B // _NW
_CH = 128
_NCH = _BPW // _CH


def _sc_gather_body(i0, i1, i2, i3, i4, t0, t1, t2, t3, t4,
                    o0, o1, o2, o3, o4,
                    tv0, tv1, tv2, tv3, tv4,
                    x0, x1, x2, x3, x4, r0, r1, r2, r3, r4,
                    s0, s1, s2, s3, s4, so):
    wid = lax.axis_index("s") * _NC + lax.axis_index("c")
    base = wid * _BPW
    idxs = (i0, i1, i2, i3, i4)
    tabs = (t0, t1, t2, t3, t4)
    outs = (o0, o1, o2, o3, o4)
    tv = (tv0, tv1, tv2, tv3, tv4)
    xb = (x0, x1, x2, x3, x4)
    rb = (r0, r1, r2, r3, r4)
    sems = (s0, s1, s2, s3, s4)
    stage = []
    for f in range(5):
        stage.append((pltpu.async_copy(tabs[f], tv[f], sems[f]),
                      pltpu.async_copy(idxs[f].at[pl.ds(base, _BPW)],
                                       xb[f], sems[f])))
    lanes = lax.iota(jnp.int32, 16)
    out_handles = []
    for f in range(5):
        stage[f][0].wait()
        stage[f][1].wait()

        def step(g, _, f=f):
            pos = g * 16 + lanes
            iv = plsc.load_gather(xb[f], [pos])
            for c in range(_DIMS[f]):
                col = jnp.full((16,), c, jnp.int32)
                val = plsc.load_gather(tv[f], [iv, col])
                plsc.store_scatter(rb[f], [pos, col], val)
            return _

        lax.fori_loop(0, _BPW // 16, step, None)
        out_handles.append(pltpu.async_copy(
            rb[f], outs[f].at[pl.ds(base, _BPW)], so))
    for h in out_handles:
        h.wait()


def _sc_gather(down_idx, form_idx, pers_idx, def_idx, sit_idx,
               E_down, E_form, E_pers, E_def, E_sit):
    mesh = plsc.VectorSubcoreMesh(core_axis_name="c", subcore_axis_name="s")
    tables = (E_down, E_form, E_pers, E_def, E_sit)
    k = pl.kernel(
        _sc_gather_body,
        [jax.ShapeDtypeStruct((B, d), jnp.float32) for d in _DIMS],
        mesh=mesh,
        compiler_params=pltpu.CompilerParams(use_tc_tiling_on_sc=False,
                                             needs_layout_passes=False),
        scratch_types=(
            [pltpu.VMEM(t.shape, jnp.float32) for t in tables]
            + [pltpu.VMEM((_BPW,), jnp.int32) for _ in range(5)]
            + [pltpu.VMEM((_BPW, d), jnp.float32) for d in _DIMS]
            + [pltpu.SemaphoreType.DMA for _ in range(6)]),
    )
    return k(down_idx, form_idx, pers_idx, def_idx, sit_idx, *tables)


def _dense_body(cont_ref, c0, c1, c2, c3, c4,
                w1_ref, b1_ref, w2_ref, b2_ref, p1_ref, b3_ref, p2_ref,
                b4_ref, out_ref):
    cont = cont_ref[...]
    p1 = p1_ref[...]
    h = jnp.maximum(cont @ w1_ref[...] + b1_ref[...], 0.0)
    ce = h @ w2_ref[...] + b2_ref[...]
    acc = ce @ p1[_OFF[5]:_OFF[6]]
    for f, cref in enumerate((c0, c1, c2, c3, c4)):
        acc = acc + cref[...] @ p1[_OFF[f]:_OFF[f + 1]]
    acc = acc + b3_ref[...]
    out_ref[...] = jnp.maximum(acc, 0.0) @ p2_ref[...] + b4_ref[...]


def kernel(continuous, down_idx, form_idx, pers_idx, def_idx, sit_idx,
           E_down, E_form, E_pers, E_def, E_sit,
           W1, b1, W2, b2, P1, b3, P2, b4):
    cat = _sc_gather(down_idx, form_idx, pers_idx, def_idx, sit_idx,
                     E_down, E_form, E_pers, E_def, E_sit)

    full = lambda shape: pl.BlockSpec(shape, lambda i: (0,) * len(shape))
    grid_spec = pl.GridSpec(
        grid=(NB,),
        in_specs=[
            pl.BlockSpec((BB, 3), lambda i: (i, 0)),
        ] + [
            pl.BlockSpec((BB, d), lambda i: (i, 0)) for d in _DIMS
        ] + [
            full((3, HID)), full((1, HID)),
            full((HID, HID // 2)), full((1, HID // 2)),
            full((280, HID)), full((1, HID)),
            full((HID, OUT)), full((1, OUT)),
        ],
        out_specs=pl.BlockSpec((BB, OUT), lambda i: (i, 0)),
    )
    return pl.pallas_call(
        _dense_body,
        grid_spec=grid_spec,
        out_shape=jax.ShapeDtypeStruct((B, OUT), jnp.float32),
    )(continuous, *cat,
      W1, b1.reshape(1, HID), W2, b2.reshape(1, HID // 2),
      P1, b3.reshape(1, HID), P2, b4.reshape(1, OUT))


# speedup vs baseline: 1.7130x; 1.2063x over previous
import jax
import jax.numpy as jnp
from jax import lax
from jax.experimental import pallas as pl
from jax.experimental.pallas import tpu as pltpu
from jax.experimental.pallas import tpu_sc as plsc

B = 16384
HID = 256
OUT = 256
BB = 4096
NB = B // BB

_DIMS = (8, 32, 32, 64, 16)
_OFF = (0, 8, 40, 72, 136, 152, 280)

_NC = 2
_NS = 16
_NW = _NC * _NS
_BPW = B // _NW
_CH = 128
_NCH = _BPW // _CH


def _sc_gather_body(i0, i1, i2, i3, i4, t0, t1, t2, t3, t4,
                    o0, o1, o2, o3, o4,
                    tv0, tv1, tv2, tv3, tv4,
                    x0, x1, x2, x3, x4, r0, r1, r2, r3, r4,
                    s0, s1, s2, s3, s4, so):
    wid = lax.axis_index("s") * _NC + lax.axis_index("c")
    base = wid * _BPW
    idxs = (i0, i1, i2, i3, i4)
    tabs = (t0, t1, t2, t3, t4)
    outs = (o0, o1, o2, o3, o4)
    tv = (tv0, tv1, tv2, tv3, tv4)
    xb = (x0, x1, x2, x3, x4)
    rb = (r0, r1, r2, r3, r4)
    sems = (s0, s1, s2, s3, s4)
    stage = []
    for f in range(5):
        stage.append((pltpu.async_copy(tabs[f], tv[f], sems[f]),
                      pltpu.async_copy(idxs[f].at[pl.ds(base, _BPW)],
                                       xb[f], sems[f])))
    lanes = lax.iota(jnp.int32, 16)
    out_handles = []
    for f in range(5):
        stage[f][0].wait()
        stage[f][1].wait()

        @plsc.parallel_loop(0, _BPW // 16, 1, unroll=4)
        def _(g, f=f):
            pos = g * 16 + lanes
            iv = plsc.load_gather(xb[f], [pos])
            for c in range(_DIMS[f]):
                col = jnp.full((16,), c, jnp.int32)
                val = plsc.load_gather(tv[f], [iv, col])
                plsc.store_scatter(rb[f], [pos, col], val)
        out_handles.append(pltpu.async_copy(
            rb[f], outs[f].at[pl.ds(base, _BPW)], so))
    for h in out_handles:
        h.wait()


def _sc_gather(down_idx, form_idx, pers_idx, def_idx, sit_idx,
               E_down, E_form, E_pers, E_def, E_sit):
    mesh = plsc.VectorSubcoreMesh(core_axis_name="c", subcore_axis_name="s")
    tables = (E_down, E_form, E_pers, E_def, E_sit)
    k = pl.kernel(
        _sc_gather_body,
        [jax.ShapeDtypeStruct((B, d), jnp.float32) for d in _DIMS],
        mesh=mesh,
        compiler_params=pltpu.CompilerParams(use_tc_tiling_on_sc=False,
                                             needs_layout_passes=False),
        scratch_types=(
            [pltpu.VMEM(t.shape, jnp.float32) for t in tables]
            + [pltpu.VMEM((_BPW,), jnp.int32) for _ in range(5)]
            + [pltpu.VMEM((_BPW, d), jnp.float32) for d in _DIMS]
            + [pltpu.SemaphoreType.DMA for _ in range(6)]),
    )
    return k(down_idx, form_idx, pers_idx, def_idx, sit_idx, *tables)


def _dense_body(cont_ref, c0, c1, c2, c3, c4,
                w1_ref, b1_ref, w2_ref, b2_ref, p1_ref, b3_ref, p2_ref,
                b4_ref, out_ref):
    cont = cont_ref[...]
    p1 = p1_ref[...]
    h = jnp.maximum(cont @ w1_ref[...] + b1_ref[...], 0.0)
    ce = h @ w2_ref[...] + b2_ref[...]
    acc = ce @ p1[_OFF[5]:_OFF[6]]
    for f, cref in enumerate((c0, c1, c2, c3, c4)):
        acc = acc + cref[...] @ p1[_OFF[f]:_OFF[f + 1]]
    acc = acc + b3_ref[...]
    out_ref[...] = jnp.maximum(acc, 0.0) @ p2_ref[...] + b4_ref[...]


def kernel(continuous, down_idx, form_idx, pers_idx, def_idx, sit_idx,
           E_down, E_form, E_pers, E_def, E_sit,
           W1, b1, W2, b2, P1, b3, P2, b4):
    cat = _sc_gather(down_idx, form_idx, pers_idx, def_idx, sit_idx,
                     E_down, E_form, E_pers, E_def, E_sit)

    full = lambda shape: pl.BlockSpec(shape, lambda i: (0,) * len(shape))
    grid_spec = pl.GridSpec(
        grid=(NB,),
        in_specs=[
            pl.BlockSpec((BB, 3), lambda i: (i, 0)),
        ] + [
            pl.BlockSpec((BB, d), lambda i: (i, 0)) for d in _DIMS
        ] + [
            full((3, HID)), full((1, HID)),
            full((HID, HID // 2)), full((1, HID // 2)),
            full((280, HID)), full((1, HID)),
            full((HID, OUT)), full((1, OUT)),
        ],
        out_specs=pl.BlockSpec((BB, OUT), lambda i: (i, 0)),
    )
    return pl.pallas_call(
        _dense_body,
        grid_spec=grid_spec,
        out_shape=jax.ShapeDtypeStruct((B, OUT), jnp.float32),
    )(continuous, *cat,
      W1, b1.reshape(1, HID), W2, b2.reshape(1, HID // 2),
      P1, b3.reshape(1, HID), P2, b4.reshape(1, OUT))
